# Initial kernel scaffold; baseline (speedup 1.0000x reference)
import jax
import jax.numpy as jnp
from jax.experimental import pallas as pl


def kernel(x, indices):
    raise NotImplementedError("write your pallas kernel here")


# speedup vs baseline: 115.0868x; 115.0868x over previous
import functools

import jax
import jax.numpy as jnp
from jax import lax
from jax.experimental import pallas as pl
from jax.experimental.pallas import tpu as pltpu
from jax.experimental.pallas import tpu_sc as plsc

B, L, D = 4, 4096, 1024
LANES = 16
LB = 16
NW = 32
ROWS_PER_W = (B * L) // NW
NBLK = ROWS_PER_W // LB
CHUNKS_PER_B = L // ROWS_PER_W


def _unpool_sc(x, indices):
    mesh = plsc.VectorSubcoreMesh(core_axis_name="c", subcore_axis_name="s")

    @functools.partial(
        pl.kernel,
        out_type=jax.ShapeDtypeStruct((B, L, 2, D), jnp.float32),
        mesh=mesh,
        compiler_params=pltpu.CompilerParams(use_tc_tiling_on_sc=False,
                                             needs_layout_passes=False),
        scratch_types=[
            pltpu.VMEM((D, LB), jnp.int32),
            pltpu.VMEM((LB, 2, D), jnp.float32),
            pltpu.SemaphoreType.DMA,
            pltpu.SemaphoreType.DMA,
        ],
    )
    def k(x_hbm, ind_hbm, out_hbm, ind_v, out_v, sem_i, sem_x):
        wid = lax.axis_index("s") * 2 + lax.axis_index("c")
        b = wid // CHUNKS_PER_B
        l0_base = (wid % CHUNKS_PER_B) * ROWS_PER_W
        iota = lax.iota(jnp.int32, LANES)

        @pl.loop(0, NBLK)
        def _(g):
            l0 = l0_base + g * LB
            cp_i = pltpu.async_copy(ind_hbm.at[b, :, pl.ds(l0, LB)], ind_v,
                                    sem_i)
            cp_x = pltpu.async_copy(x_hbm.at[b, pl.ds(l0, LB), :],
                                    out_v.at[:, 0, :], sem_x)
            cp_i.wait()
            cp_x.wait()

            @pl.loop(0, D // LANES)
            def _(dv):
                rowv = dv * LANES + iota
                for l in range(LB):
                    colv = jnp.full((LANES,), l, jnp.int32)
                    gi = plsc.load_gather(ind_v, [rowv, colv])
                    m = (gi & 1) == 1
                    v = out_v[l, 0, pl.ds(dv * LANES, LANES)]
                    out_v[l, 1, pl.ds(dv * LANES, LANES)] = jnp.where(
                        m, v, 0.0)
                    out_v[l, 0, pl.ds(dv * LANES, LANES)] = jnp.where(
                        m, 0.0, v)

            pltpu.sync_copy(out_v, out_hbm.at[b, pl.ds(l0, LB)])

    return k(x, indices)


def kernel(x, indices):
    out = _unpool_sc(x, indices)
    return out.reshape(B, 2 * L, D)


# speedup vs baseline: 224.9432x; 1.9546x over previous
import functools

import jax
import jax.numpy as jnp
from jax import lax
from jax.experimental import pallas as pl
from jax.experimental.pallas import tpu as pltpu
from jax.experimental.pallas import tpu_sc as plsc

B, L, D = 4, 4096, 1024
LANES = 16
LB = 16
NW = 32
ROWS_PER_W = (B * L) // NW
NBLK = ROWS_PER_W // LB
CHUNKS_PER_B = L // ROWS_PER_W


def _unpool_sc(x, indices):
    mesh = plsc.VectorSubcoreMesh(core_axis_name="c", subcore_axis_name="s")

    @functools.partial(
        pl.kernel,
        out_type=jax.ShapeDtypeStruct((B, L, 2, D), jnp.float32),
        mesh=mesh,
        compiler_params=pltpu.CompilerParams(use_tc_tiling_on_sc=False,
                                             needs_layout_passes=False),
        scratch_types=[
            pltpu.VMEM((D, LB), jnp.int32),
            pltpu.VMEM((LB, D), jnp.float32),
            pltpu.VMEM((LB, D), jnp.float32),
            pltpu.VMEM((LB, 2, D), jnp.float32),
            pltpu.VMEM((LB, 2, D), jnp.float32),
            pltpu.SemaphoreType.DMA,
            pltpu.SemaphoreType.DMA,
            pltpu.SemaphoreType.DMA,
            pltpu.SemaphoreType.DMA,
            pltpu.SemaphoreType.DMA,
        ],
    )
    def k(x_hbm, ind_hbm, out_hbm, ind_v, x_v0, x_v1, out_v0, out_v1,
          si, sx0, sx1, so0, so1):
        wid = lax.axis_index("s") * 2 + lax.axis_index("c")
        b = wid // CHUNKS_PER_B
        l0_base = (wid % CHUNKS_PER_B) * ROWS_PER_W
        iota = lax.iota(jnp.int32, LANES)

        def ind_cp(g):
            return pltpu.make_async_copy(
                ind_hbm.at[b, :, pl.ds(l0_base + g * LB, LB)], ind_v, si)

        def x_cp(g, xv, sem):
            return pltpu.make_async_copy(
                x_hbm.at[b, pl.ds(l0_base + g * LB, LB), :], xv, sem)

        def out_cp(g, ov, sem):
            return pltpu.make_async_copy(
                ov, out_hbm.at[b, pl.ds(l0_base + g * LB, LB)], sem)

        def compute(xv, ov):
            @functools.partial(plsc.parallel_loop, 0, D // LANES, unroll=2)
            def _(dv):
                rowv = dv * LANES + iota
                base = dv * LANES
                for l in range(LB):
                    colv = jnp.full((LANES,), l, jnp.int32)
                    gi = plsc.load_gather(ind_v, [rowv, colv])
                    m = (gi & 1) == 1
                    v = xv[l, pl.ds(base, LANES)]
                    ov[l, 1, pl.ds(base, LANES)] = jnp.where(m, v, 0.0)
                    ov[l, 0, pl.ds(base, LANES)] = jnp.where(m, 0.0, v)

        ind_cp(0).start()
        x_cp(0, x_v0, sx0).start()
        x_cp(1, x_v1, sx1).start()

        slots = ((x_v0, sx0, out_v0, so0), (x_v1, sx1, out_v1, so1))

        @pl.loop(0, NBLK // 2)
        def _(gp):
            for s, (xv, sx, ov, so) in enumerate(slots):
                g = gp * 2 + s

                ind_cp(g).wait()
                x_cp(g, xv, sx).wait()

                @pl.when(g >= 2)
                def _():
                    out_cp(g - 2, ov, so).wait()

                compute(xv, ov)
                out_cp(g, ov, so).start()

                @pl.when(g + 1 < NBLK)
                def _():
                    ind_cp(g + 1).start()

                @pl.when(g + 2 < NBLK)
                def _():
                    x_cp(g + 2, xv, sx).start()

        out_cp(NBLK - 2, out_v0, so0).wait()
        out_cp(NBLK - 1, out_v1, so1).wait()

    return k(x, indices)


def kernel(x, indices):
    out = _unpool_sc(x, indices)
    return out.reshape(B, 2 * L, D)


# speedup vs baseline: 723.2472x; 3.2152x over previous
import functools

import jax
import jax.numpy as jnp
from jax import lax
from jax.experimental import pallas as pl
from jax.experimental.pallas import tpu as pltpu
from jax.experimental.pallas import tpu_sc as plsc

B, L, D = 4, 4096, 1024
LANES = 16
NW = 32
ROWS_PER_W = (B * L) // NW
LBLK = 128
DBLK = 128
NLB = ROWS_PER_W // LBLK
NDQ = D // DBLK
NBLK = NLB * NDQ
CHUNKS_PER_B = L // ROWS_PER_W


def _unpool_sc(x4, ind4):
    mesh = plsc.VectorSubcoreMesh(core_axis_name="c", subcore_axis_name="s")

    @functools.partial(
        pl.kernel,
        out_type=jax.ShapeDtypeStruct((B, 2 * L // 8, 8, D), jnp.float32),
        mesh=mesh,
        scratch_types=[
            pltpu.VMEM((16, 8, DBLK), jnp.int32),
            pltpu.VMEM((16, 8, DBLK), jnp.float32),
            pltpu.VMEM((16, 8, DBLK), jnp.float32),
            pltpu.VMEM((32, 8, DBLK), jnp.float32),
            pltpu.VMEM((32, 8, DBLK), jnp.float32),
            pltpu.SemaphoreType.DMA,
            pltpu.SemaphoreType.DMA,
            pltpu.SemaphoreType.DMA,
            pltpu.SemaphoreType.DMA,
            pltpu.SemaphoreType.DMA,
        ],
    )
    def k(x_hbm, ind_hbm, out_hbm, ind_v, x_v0, x_v1, out_v0, out_v1,
          si, sx0, sx1, so0, so1):
        wid = lax.axis_index("s") * 2 + lax.axis_index("c")
        b = wid // CHUNKS_PER_B
        l0_base = (wid % CHUNKS_PER_B) * ROWS_PER_W
        iota = lax.iota(jnp.int32, LANES)
        hi8 = iota // 8
        lo8 = iota % 8

        def split(g):
            return g // NDQ, g % NDQ

        def ind_cp(g):
            lb, dq = split(g)
            return pltpu.make_async_copy(
                ind_hbm.at[b, pl.ds(dq * 16, 16), :,
                           pl.ds(l0_base + lb * LBLK, LBLK)],
                ind_v, si)

        def x_cp(g, xv, sem):
            lb, dq = split(g)
            return pltpu.make_async_copy(
                x_hbm.at[b, pl.ds(l0_base // 8 + lb * 16, 16), :,
                         pl.ds(dq * DBLK, DBLK)],
                xv, sem)

        def out_cp(g, ov, sem):
            lb, dq = split(g)
            return pltpu.make_async_copy(
                ov,
                out_hbm.at[b, pl.ds(l0_base // 4 + lb * 32, 32), :,
                           pl.ds(dq * DBLK, DBLK)],
                sem)

        def compute(xv, ov):
            @functools.partial(plsc.parallel_loop, 0, LBLK // 8)
            def _(lr):
                lr8 = lr * 8
                for ls in range(8):
                    lvec = jnp.full((LANES,), lr8 + ls, jnp.int32)
                    for dv in range(DBLK // LANES):
                        gi = plsc.load_gather(
                            ind_v, [dv * 2 + hi8, lo8, lvec])
                        m = (gi & 1) == 1
                        v = xv[lr, ls, pl.ds(dv * LANES, LANES)]
                        r = 2 * ls + 1
                        ov[2 * lr + r // 8, r % 8,
                           pl.ds(dv * LANES, LANES)] = jnp.where(m, v, 0.0)
                        r = 2 * ls
                        ov[2 * lr + r // 8, r % 8,
                           pl.ds(dv * LANES, LANES)] = jnp.where(m, 0.0, v)

        ind_cp(0).start()
        x_cp(0, x_v0, sx0).start()
        x_cp(1, x_v1, sx1).start()

        slots = ((x_v0, sx0, out_v0, so0), (x_v1, sx1, out_v1, so1))

        @pl.loop(0, NBLK // 2)
        def _(gp):
            for s, (xv, sx, ov, so) in enumerate(slots):
                g = gp * 2 + s

                ind_cp(g).wait()
                x_cp(g, xv, sx).wait()

                @pl.when(g >= 2)
                def _():
                    out_cp(g - 2, ov, so).wait()

                compute(xv, ov)
                out_cp(g, ov, so).start()

                @pl.when(g + 1 < NBLK)
                def _():
                    ind_cp(g + 1).start()

                @pl.when(g + 2 < NBLK)
                def _():
                    x_cp(g + 2, xv, sx).start()

        out_cp(NBLK - 2, out_v0, so0).wait()
        out_cp(NBLK - 1, out_v1, so1).wait()

    return k(x4, ind4)


def kernel(x, indices):
    x4 = x.reshape(B, L // 8, 8, D)
    ind4 = indices.reshape(B, D // 8, 8, L)
    out = _unpool_sc(x4, ind4)
    return out.reshape(B, 2 * L, D)


# speedup vs baseline: 725.5562x; 1.0032x over previous
import functools

import jax
import jax.numpy as jnp
from jax import lax
from jax.experimental import pallas as pl
from jax.experimental.pallas import tpu as pltpu
from jax.experimental.pallas import tpu_sc as plsc

B, L, D = 4, 4096, 1024
LANES = 16
NW = 32
ROWS_PER_W = (B * L) // NW
LBLK = 128
DBLK = 128
NLB = ROWS_PER_W // LBLK
NDQ = D // DBLK
NBLK = NLB * NDQ
CHUNKS_PER_B = L // ROWS_PER_W


def _unpool_sc(x4, ind4):
    mesh = plsc.VectorSubcoreMesh(core_axis_name="c", subcore_axis_name="s")

    @functools.partial(
        pl.kernel,
        out_type=jax.ShapeDtypeStruct((B, 2 * L // 8, 8, D), jnp.float32),
        mesh=mesh,
        scratch_types=[
            pltpu.VMEM((8, 8, DBLK), jnp.int32),
            pltpu.VMEM((8, 8, DBLK), jnp.int32),
            pltpu.VMEM((16, 8, DBLK), jnp.float32),
            pltpu.VMEM((16, 8, DBLK), jnp.float32),
            pltpu.VMEM((32, 8, DBLK), jnp.float32),
            pltpu.VMEM((32, 8, DBLK), jnp.float32),
            pltpu.SemaphoreType.DMA,
            pltpu.SemaphoreType.DMA,
            pltpu.SemaphoreType.DMA,
            pltpu.SemaphoreType.DMA,
            pltpu.SemaphoreType.DMA,
            pltpu.SemaphoreType.DMA,
        ],
    )
    def k(x_hbm, ind_hbm, out_hbm, ind_a, ind_b, x_v0, x_v1, out_v0,
          out_v1, sia, sib, sx0, sx1, so0, so1):
        wid = lax.axis_index("s") * 2 + lax.axis_index("c")
        b = wid // CHUNKS_PER_B
        l0_base = (wid % CHUNKS_PER_B) * ROWS_PER_W
        iota = lax.iota(jnp.int32, LANES)
        hi8 = iota // 8
        lo8 = iota % 8

        def split(g):
            return g // NDQ, g % NDQ

        def ind_cp(g, h):
            lb, dq = split(g)
            buf, sem = (ind_a, sia) if h == 0 else (ind_b, sib)
            return pltpu.make_async_copy(
                ind_hbm.at[b, pl.ds(dq * 16 + 8 * h, 8), :,
                           pl.ds(l0_base + lb * LBLK, LBLK)],
                buf, sem)

        def x_cp(g, xv, sem):
            lb, dq = split(g)
            return pltpu.make_async_copy(
                x_hbm.at[b, pl.ds(l0_base // 8 + lb * 16, 16), :,
                         pl.ds(dq * DBLK, DBLK)],
                xv, sem)

        def out_cp(g, ov, sem):
            lb, dq = split(g)
            return pltpu.make_async_copy(
                ov,
                out_hbm.at[b, pl.ds(l0_base // 4 + lb * 32, 32), :,
                           pl.ds(dq * DBLK, DBLK)],
                sem)

        def compute(xv, ov, h):
            buf = ind_a if h == 0 else ind_b
            @functools.partial(plsc.parallel_loop, 0, LBLK // 8)
            def _(lr):
                lr8 = lr * 8
                for ls in range(8):
                    lvec = jnp.full((LANES,), lr8 + ls, jnp.int32)
                    for dv in range(4 * h, 4 * h + 4):
                        gi = plsc.load_gather(
                            buf, [(dv - 4 * h) * 2 + hi8, lo8, lvec])
                        m = (gi & 1) == 1
                        v = xv[lr, ls, pl.ds(dv * LANES, LANES)]
                        r = 2 * ls + 1
                        ov[2 * lr + r // 8, r % 8,
                           pl.ds(dv * LANES, LANES)] = jnp.where(m, v, 0.0)
                        r = 2 * ls
                        ov[2 * lr + r // 8, r % 8,
                           pl.ds(dv * LANES, LANES)] = jnp.where(m, 0.0, v)

        ind_cp(0, 0).start()
        ind_cp(0, 1).start()
        x_cp(0, x_v0, sx0).start()
        x_cp(1, x_v1, sx1).start()

        slots = ((x_v0, sx0, out_v0, so0), (x_v1, sx1, out_v1, so1))

        @pl.loop(0, NBLK // 2)
        def _(gp):
            for s, (xv, sx, ov, so) in enumerate(slots):
                g = gp * 2 + s

                ind_cp(g, 0).wait()
                ind_cp(g, 1).wait()
                x_cp(g, xv, sx).wait()

                @pl.when(g >= 2)
                def _():
                    out_cp(g - 2, ov, so).wait()

                compute(xv, ov, 0)
                compute(xv, ov, 1)
                out_cp(g, ov, so).start()

                @pl.when(g + 1 < NBLK)
                def _():
                    ind_cp(g + 1, 0).start()
                    ind_cp(g + 1, 1).start()

                @pl.when(g + 2 < NBLK)
                def _():
                    x_cp(g + 2, xv, sx).start()

        out_cp(NBLK - 2, out_v0, so0).wait()
        out_cp(NBLK - 1, out_v1, so1).wait()

    return k(x4, ind4)


def kernel(x, indices):
    x4 = x.reshape(B, L // 8, 8, D)
    ind4 = indices.reshape(B, D // 8, 8, L)
    out = _unpool_sc(x4, ind4)
    return out.reshape(B, 2 * L, D)
